# Initial kernel scaffold; baseline (speedup 1.0000x reference)
import jax
import jax.numpy as jnp
from jax.experimental import pallas as pl


def kernel(x, edge_index, edge_attr, batch, W_e, b_e, W1, b1, W2, b2, Wf1, bf1, Wf2, bf2):
    raise NotImplementedError("write your pallas kernel here")


# speedup vs baseline: 9.8386x; 9.8386x over previous
import functools

import jax
import jax.numpy as jnp
from jax import lax
from jax.experimental import pallas as pl
from jax.experimental.pallas import tpu as pltpu
from jax.experimental.pallas import tpu_sc as plsc

N = 100000
E = 3200000
G = 1024
D = 16
NC = 2
NS = 16
NW = NC * NS
EW = E // NW
SUP = 400
SUB = 80
NSUB = SUP // SUB
NSUPER = EW // SUP
NGRP = SUP // 16
NP = 100352
ROWS_W = NP // NS

RB = 1000
NBLK = N // RB


def _edge_sc_kernel(x_hbm, src_hbm, dst_hbm, ea0_hbm, ea1_hbm, ea2_hbm,
                    we_hbm, be_hbm, zeros_hbm, out_hbm,
                    src_v, dst_v, ea0_v, ea1_v, ea2_v, rows_v, we_v, be_v,
                    aggr_sh, sem):
    c = lax.axis_index("c")
    s = lax.axis_index("s")
    wid = s * NC + c

    pltpu.sync_copy(zeros_hbm.at[pl.ds(s * ROWS_W, ROWS_W)],
                    aggr_sh.at[pl.ds(s * ROWS_W, ROWS_W)])
    pltpu.sync_copy(we_hbm, we_v)
    pltpu.sync_copy(be_hbm, be_v)
    plsc.subcore_barrier()

    wrows = [we_v[k] for k in range(3)]
    brow = be_v[...]
    w = [[wrows[k][f] for f in range(5)] for k in range(3)]
    b = [brow[f] for f in range(5)]
    iota16 = lax.broadcasted_iota(jnp.int32, (16,), 0)
    cols = [jnp.full((16,), f, jnp.int32) for f in range(5)]

    def super_body(sc, carry):
        base = wid * EW + sc * SUP
        pltpu.sync_copy(src_hbm.at[pl.ds(base, SUP)], src_v)
        pltpu.sync_copy(dst_hbm.at[pl.ds(wid * (EW // SUB) + sc * NSUB, NSUB)],
                        dst_v)
        pltpu.sync_copy(ea0_hbm.at[pl.ds(base, SUP)], ea0_v)
        pltpu.sync_copy(ea1_hbm.at[pl.ds(base, SUP)], ea1_v)
        pltpu.sync_copy(ea2_hbm.at[pl.ds(base, SUP)], ea2_v)
        descs = []
        for j in range(NSUB):
            descs.append(pltpu.async_copy(
                x_hbm.at[src_v.at[pl.ds(j * SUB, SUB)]],
                rows_v.at[pl.ds(j * SUB, SUB)], sem))
        for d in descs:
            d.wait()

        def group_body(g, carry2):
            ridx = iota16 + g * 16
            ea0 = ea0_v[pl.ds(g * 16, 16)]
            ea1 = ea1_v[pl.ds(g * 16, 16)]
            ea2 = ea2_v[pl.ds(g * 16, 16)]
            for f in range(5):
                xg = plsc.load_gather(rows_v, [ridx, cols[f]])
                m = xg + ea0 * w[0][f] + ea1 * w[1][f] + ea2 * w[2][f] + b[f]
                plsc.store_scatter(rows_v, [ridx, cols[f]],
                                   jnp.maximum(m, 0.0))
            return carry2

        lax.fori_loop(0, NGRP, group_body, 0)

        for j in range(NSUB):
            pltpu.sync_copy(rows_v.at[pl.ds(j * SUB, SUB)],
                            aggr_sh.at[dst_v.at[j]], add=True)
        return carry

    lax.fori_loop(0, NSUPER, super_body, 0)
    plsc.subcore_barrier()
    pltpu.sync_copy(aggr_sh.at[pl.ds(s * ROWS_W, ROWS_W)],
                    out_hbm.at[c, pl.ds(s * ROWS_W, ROWS_W)])


def _make_edge_kernel():
    mesh = plsc.VectorSubcoreMesh(core_axis_name="c", subcore_axis_name="s",
                                  num_cores=NC, num_subcores=NS)
    return pl.kernel(
        _edge_sc_kernel,
        out_type=jax.ShapeDtypeStruct((NC, NP, D), jnp.float32),
        mesh=mesh,
        scratch_types=[
            pltpu.VMEM((SUP,), jnp.int32),
            pltpu.VMEM((NSUB, SUB), jnp.int32),
            pltpu.VMEM((SUP,), jnp.float32),
            pltpu.VMEM((SUP,), jnp.float32),
            pltpu.VMEM((SUP,), jnp.float32),
            pltpu.VMEM((SUP, D), jnp.float32),
            pltpu.VMEM((3, D), jnp.float32),
            pltpu.VMEM((D,), jnp.float32),
            pltpu.VMEM_SHARED((NP, D), jnp.float32),
            pltpu.SemaphoreType.DMA,
        ],
        compiler_params=pltpu.CompilerParams(use_tc_tiling_on_sc=False,
                                             needs_layout_passes=False),
    )


def _tc_body(xp_ref, a0_ref, a1_ref, batch_ref, w1_ref, b1_ref, w2_ref,
             b2_ref, wf1_ref, bf1_ref, wf2_ref, bf2_ref, out_ref, acc_ref):
    i = pl.program_id(0)

    @pl.when(i == 0)
    def _():
        acc_ref[...] = jnp.zeros_like(acc_ref)

    hin = xp_ref[...] + a0_ref[0] + a1_ref[0]
    t = jnp.maximum(
        lax.dot_general(hin, w1_ref[...], (((1,), (0,)), ((), ()))) +
        b1_ref[...], 0.0)
    h = lax.dot_general(t, w2_ref[...], (((1,), (0,)), ((), ()))) + b2_ref[...]
    gids = lax.broadcasted_iota(jnp.int32, (G, RB), 0)
    onehot = (gids == batch_ref[0]).astype(jnp.float32)
    acc_ref[...] += lax.dot_general(onehot, h, (((1,), (0,)), ((), ())),
                                    precision=lax.Precision.HIGHEST)

    @pl.when(i == NBLK - 1)
    def _():
        sums = acc_ref[...]
        cnt = jnp.maximum(sums[:, 15:16], 1.0)
        mean = sums / cnt
        z = jnp.maximum(
            lax.dot_general(mean, wf1_ref[...], (((1,), (0,)), ((), ()))) +
            bf1_ref[...], 0.0)
        o = lax.dot_general(z, wf2_ref[...], (((1,), (0,)), ((), ()))) + bf2_ref[...]
        out_ref[...] = jax.nn.sigmoid(o)


def _pad2(a, rows, cols):
    return jnp.zeros((rows, cols), a.dtype).at[:a.shape[0], :a.shape[1]].set(a)


def kernel(x, edge_index, edge_attr, batch, W_e, b_e, W1, b1, W2, b2,
           Wf1, bf1, Wf2, bf2):
    src = edge_index[0].astype(jnp.int32)
    dst = edge_index[1].astype(jnp.int32)
    xp = jnp.zeros((NP, D), jnp.float32).at[:N, :5].set(x)
    eab = edge_attr.astype(jnp.bfloat16).astype(jnp.float32)
    ea0 = eab[:, 0]
    ea1 = eab[:, 1]
    ea2 = eab[:, 2]
    wep = _pad2(W_e, 3, D).astype(jnp.bfloat16).astype(jnp.float32)
    bep = jnp.zeros((D,), jnp.float32).at[:5].set(b_e)
    zeros = jnp.zeros((NP, D), jnp.float32)

    aggr = _make_edge_kernel()(xp, src, dst.reshape(E // SUB, SUB),
                               ea0, ea1, ea2, wep, bep, zeros)
    aggr = aggr[:, :N, :]
    xpn = xp[:N, :]

    w1p = _pad2(W1, D, 128)
    b1p = b1.reshape(1, 128)
    w2p = _pad2(W2, 128, D)
    b2p = jnp.zeros((1, D), jnp.float32).at[0, :5].set(b2).at[0, 15].set(1.0)
    wf1p = _pad2(Wf1, D, 128)
    bf1p = jnp.zeros((1, 128), jnp.float32).at[0, :32].set(bf1)
    wf2p = _pad2(Wf2, 128, 128)
    bf2p = jnp.zeros((1, 128), jnp.float32).at[0, :1].set(bf2)
    batch3d = batch.astype(jnp.int32).reshape(NBLK, 1, RB)

    full = lambda shape: pl.BlockSpec(shape, lambda i: tuple(0 for _ in shape))
    out = pl.pallas_call(
        _tc_body,
        grid=(NBLK,),
        in_specs=[
            pl.BlockSpec((RB, D), lambda i: (i, 0)),
            pl.BlockSpec((1, RB, D), lambda i: (0, i, 0)),
            pl.BlockSpec((1, RB, D), lambda i: (1, i, 0)),
            pl.BlockSpec((1, 1, RB), lambda i: (i, 0, 0)),
            full((D, 128)), full((1, 128)), full((128, D)), full((1, D)),
            full((D, 128)), full((1, 128)), full((128, 128)), full((1, 128)),
        ],
        out_specs=pl.BlockSpec((G, 128), lambda i: (0, 0)),
        out_shape=jax.ShapeDtypeStruct((G, 128), jnp.float32),
        scratch_shapes=[pltpu.VMEM((G, D), jnp.float32)],
    )(xpn, aggr, aggr, batch3d, w1p, b1p, w2p, b2p, wf1p, bf1p, wf2p, bf2p)

    return out[:, :1]


# speedup vs baseline: 13.5636x; 1.3786x over previous
import functools

import jax
import jax.numpy as jnp
from jax import lax
from jax.experimental import pallas as pl
from jax.experimental.pallas import tpu as pltpu
from jax.experimental.pallas import tpu_sc as plsc

N = 100000
E = 3200000
G = 1024
D = 16
NC = 2
NS = 16
NW = NC * NS
EW = E // NW
SUP = 2000
SUB = 80
NSUB = SUP // SUB
NSUPER = EW // SUP
NGRP = SUP // 16
NP = 100352
ROWS_W = NP // NS

DA = 8
RB = 1000
NBLK = N // RB


def _edge_sc_kernel(x_hbm, src_hbm, dst_hbm, ea0_hbm, ea1_hbm, ea2_hbm,
                    we_hbm, be_hbm, zeros_hbm, out_hbm,
                    src_v, dst_v, ea0_v, ea1_v, ea2_v, rows_v, msg_v, we_v,
                    be_v, aggr_sh, sem):
    c = lax.axis_index("c")
    s = lax.axis_index("s")
    wid = s * NC + c

    pltpu.sync_copy(zeros_hbm.at[pl.ds(s * ROWS_W, ROWS_W)],
                    aggr_sh.at[pl.ds(s * ROWS_W, ROWS_W)])
    pltpu.sync_copy(zeros_hbm.at[pl.ds(0, SUP)], msg_v)
    pltpu.sync_copy(we_hbm, we_v)
    pltpu.sync_copy(be_hbm, be_v)
    plsc.subcore_barrier()

    wrows = [we_v[k] for k in range(3)]
    brow = be_v[...]
    w = [[wrows[k][f] for f in range(5)] for k in range(3)]
    b = [brow[f] for f in range(5)]
    iota16 = lax.broadcasted_iota(jnp.int32, (16,), 0)
    cols = [jnp.full((16,), f, jnp.int32) for f in range(5)]

    def super_body(sc, carry):
        base = wid * EW + sc * SUP
        pltpu.sync_copy(src_hbm.at[pl.ds(base, SUP)], src_v)
        pltpu.sync_copy(dst_hbm.at[pl.ds(wid * (EW // SUB) + sc * NSUB, NSUB)],
                        dst_v)
        pltpu.sync_copy(ea0_hbm.at[pl.ds(base, SUP)], ea0_v)
        pltpu.sync_copy(ea1_hbm.at[pl.ds(base, SUP)], ea1_v)
        pltpu.sync_copy(ea2_hbm.at[pl.ds(base, SUP)], ea2_v)
        descs = []
        for j in range(NSUB):
            descs.append(pltpu.async_copy(
                x_hbm.at[src_v.at[pl.ds(j * SUB, SUB)]],
                rows_v.at[pl.ds(j * SUB, SUB)], sem))
        for d in descs:
            d.wait()

        def group_body(g, carry2):
            ridx = iota16 + g * 16
            ea0 = ea0_v[pl.ds(g * 16, 16)]
            ea1 = ea1_v[pl.ds(g * 16, 16)]
            ea2 = ea2_v[pl.ds(g * 16, 16)]
            for f in range(5):
                xg = plsc.load_gather(rows_v, [ridx, cols[f]])
                m = xg + ea0 * w[0][f] + ea1 * w[1][f] + ea2 * w[2][f] + b[f]
                plsc.store_scatter(msg_v, [ridx, cols[f]],
                                   jnp.maximum(m, 0.0))
            return carry2

        lax.fori_loop(0, NGRP, group_body, 0)

        for j in range(NSUB):
            pltpu.sync_copy(msg_v.at[pl.ds(j * SUB, SUB)],
                            aggr_sh.at[dst_v.at[j]], add=True)
        return carry

    lax.fori_loop(0, NSUPER, super_body, 0)
    plsc.subcore_barrier()
    pltpu.sync_copy(aggr_sh.at[pl.ds(s * ROWS_W, ROWS_W)],
                    out_hbm.at[c, pl.ds(s * ROWS_W, ROWS_W)])


def _make_edge_kernel():
    mesh = plsc.VectorSubcoreMesh(core_axis_name="c", subcore_axis_name="s",
                                  num_cores=NC, num_subcores=NS)
    return pl.kernel(
        _edge_sc_kernel,
        out_type=jax.ShapeDtypeStruct((NC, NP, DA), jnp.float32),
        mesh=mesh,
        scratch_types=[
            pltpu.VMEM((SUP,), jnp.int32),
            pltpu.VMEM((NSUB, SUB), jnp.int32),
            pltpu.VMEM((SUP,), jnp.float32),
            pltpu.VMEM((SUP,), jnp.float32),
            pltpu.VMEM((SUP,), jnp.float32),
            pltpu.VMEM((SUP, D), jnp.float32),
            pltpu.VMEM((SUP, DA), jnp.float32),
            pltpu.VMEM((3, D), jnp.float32),
            pltpu.VMEM((D,), jnp.float32),
            pltpu.VMEM_SHARED((NP, DA), jnp.float32),
            pltpu.SemaphoreType.DMA,
        ],
        compiler_params=pltpu.CompilerParams(use_tc_tiling_on_sc=False,
                                             needs_layout_passes=False),
    )


def _tc_body(xp_ref, a0_ref, a1_ref, batch_ref, w1_ref, b1_ref, w2_ref,
             b2_ref, wf1_ref, bf1_ref, wf2_ref, bf2_ref, out_ref, acc_ref):
    i = pl.program_id(0)

    @pl.when(i == 0)
    def _():
        acc_ref[...] = jnp.zeros_like(acc_ref)

    def _bf(v):
        return v.astype(jnp.bfloat16).astype(jnp.float32)

    hin = xp_ref[...] + a0_ref[0] + a1_ref[0]
    t = jnp.maximum(
        lax.dot_general(_bf(hin), w1_ref[...], (((1,), (0,)), ((), ()))) +
        b1_ref[...], 0.0)
    h = lax.dot_general(_bf(t), w2_ref[...], (((1,), (0,)), ((), ()))) + b2_ref[...]
    gids = lax.broadcasted_iota(jnp.int32, (G, RB), 0)
    onehot = (gids == batch_ref[0]).astype(jnp.float32)
    acc_ref[...] += lax.dot_general(onehot, h, (((1,), (0,)), ((), ())),
                                    precision=lax.Precision.HIGHEST)

    @pl.when(i == NBLK - 1)
    def _():
        sums = acc_ref[...]
        cnt = jnp.maximum(sums[:, DA - 1:DA], 1.0)
        mean = sums / cnt
        z = jnp.maximum(
            lax.dot_general(_bf(mean), wf1_ref[...], (((1,), (0,)), ((), ()))) +
            bf1_ref[...], 0.0)
        o = lax.dot_general(_bf(z), wf2_ref[...], (((1,), (0,)), ((), ()))) + bf2_ref[...]
        out_ref[...] = jax.nn.sigmoid(o)


def _pad2(a, rows, cols):
    return jnp.zeros((rows, cols), a.dtype).at[:a.shape[0], :a.shape[1]].set(a)


def kernel(x, edge_index, edge_attr, batch, W_e, b_e, W1, b1, W2, b2,
           Wf1, bf1, Wf2, bf2):
    src = edge_index[0].astype(jnp.int32)
    dst = edge_index[1].astype(jnp.int32)
    xp = jnp.zeros((NP, D), jnp.float32).at[:N, :5].set(x)
    eab = edge_attr.astype(jnp.bfloat16).astype(jnp.float32)
    ea0 = eab[:, 0]
    ea1 = eab[:, 1]
    ea2 = eab[:, 2]
    wep = _pad2(W_e, 3, D).astype(jnp.bfloat16).astype(jnp.float32)
    bep = jnp.zeros((D,), jnp.float32).at[:5].set(b_e)
    zeros = jnp.zeros((NP, DA), jnp.float32)

    aggr = _make_edge_kernel()(xp, src, dst.reshape(E // SUB, SUB),
                               ea0, ea1, ea2, wep, bep, zeros)
    aggr = aggr[:, :N, :]
    xpn = jnp.zeros((N, DA), jnp.float32).at[:, :5].set(x)

    rbf = lambda a: a.astype(jnp.bfloat16).astype(jnp.float32)
    w1p = rbf(_pad2(W1, DA, 128))
    b1p = b1.reshape(1, 128)
    w2p = rbf(_pad2(W2, 128, DA))
    b2p = jnp.zeros((1, DA), jnp.float32).at[0, :5].set(b2).at[0, DA - 1].set(1.0)
    wf1p = rbf(_pad2(Wf1, DA, 128))
    bf1p = jnp.zeros((1, 128), jnp.float32).at[0, :32].set(bf1)
    wf2p = rbf(_pad2(Wf2, 128, 128))
    bf2p = jnp.zeros((1, 128), jnp.float32).at[0, :1].set(bf2)
    batch3d = batch.astype(jnp.int32).reshape(NBLK, 1, RB)

    full = lambda shape: pl.BlockSpec(shape, lambda i: tuple(0 for _ in shape))
    out = pl.pallas_call(
        _tc_body,
        grid=(NBLK,),
        in_specs=[
            pl.BlockSpec((RB, DA), lambda i: (i, 0)),
            pl.BlockSpec((1, RB, DA), lambda i: (0, i, 0)),
            pl.BlockSpec((1, RB, DA), lambda i: (1, i, 0)),
            pl.BlockSpec((1, 1, RB), lambda i: (i, 0, 0)),
            full((DA, 128)), full((1, 128)), full((128, DA)), full((1, DA)),
            full((DA, 128)), full((1, 128)), full((128, 128)), full((1, 128)),
        ],
        out_specs=pl.BlockSpec((G, 128), lambda i: (0, 0)),
        out_shape=jax.ShapeDtypeStruct((G, 128), jnp.float32),
        scratch_shapes=[pltpu.VMEM((G, DA), jnp.float32)],
    )(xpn, aggr, aggr, batch3d, w1p, b1p, w2p, b2p, wf1p, bf1p, wf2p, bf2p)

    return out[:, :1]
